# Initial kernel scaffold; baseline (speedup 1.0000x reference)
import jax
import jax.numpy as jnp
from jax.experimental import pallas as pl


def kernel(x, edge_index, edge_inform, batch, params):
    raise NotImplementedError("write your pallas kernel here")


# speedup vs baseline: 1.1151x; 1.1151x over previous
import functools

import jax
import jax.numpy as jnp
from jax import lax
from jax.experimental import pallas as pl
from jax.experimental.pallas import tpu as pltpu
from jax.experimental.pallas import tpu_sc as plsc

N = 10000
E = 320000
B = 64
DH = 128
LAYERS = 6

NC = 2
NS = 16
LANES = 16

EB = 2560
C = 80
NBUF = 2
EPT = E // (NC * NS)
NCHUNK = EPT // C
NSUPER = NCHUNK // NBUF
TROWS = 624
ZROWS = 8
NB = 2000
NGRID = N // NB


def _gates_body(e_ref, *refs):
    wrefs = refs[:3 * LAYERS]
    orefs = refs[3 * LAYERS:]
    eb = e_ref[...]
    for l in range(LAYERS):
        we, be, wg = wrefs[3 * l], wrefs[3 * l + 1], wrefs[3 * l + 2]
        eh = jnp.maximum(
            jnp.dot(eb, we[...], preferred_element_type=jnp.float32) + be[...], 0.0)
        orefs[l][...] = jax.nn.sigmoid(
            jnp.dot(eh, wg[...], preferred_element_type=jnp.float32))
        eb = eh


def _compute_gates(e, blocks):
    ins = [e]
    in_specs = [pl.BlockSpec((EB, e.shape[1]), lambda i: (i, 0))]
    for blk in blocks:
        for w in (blk["We"], blk["be"].reshape(1, -1), blk["Wg"]):
            ins.append(w)
            in_specs.append(pl.BlockSpec(w.shape, lambda i: (0, 0)))
    return pl.pallas_call(
        _gates_body,
        grid=(E // EB,),
        in_specs=in_specs,
        out_specs=[pl.BlockSpec((EB, DH), lambda i: (i, 0))] * LAYERS,
        out_shape=[jax.ShapeDtypeStruct((E, DH), jnp.float32)] * LAYERS,
    )(*ins)


def _node_mm_body(h_ref, wn_ref, ws_ref, bn_ref, hw_ref, hs_ref):
    h = h_ref[...]
    hw_ref[...] = jnp.dot(h, wn_ref[...], preferred_element_type=jnp.float32)
    hs_ref[...] = (
        jnp.dot(h, ws_ref[...], preferred_element_type=jnp.float32) + bn_ref[...])


def _node_mm(h, wn, ws, bn):
    d_in = h.shape[1]
    return pl.pallas_call(
        _node_mm_body,
        grid=(NGRID,),
        in_specs=[
            pl.BlockSpec((NB, d_in), lambda i: (i, 0)),
            pl.BlockSpec((d_in, DH), lambda i: (0, 0)),
            pl.BlockSpec((d_in, DH), lambda i: (0, 0)),
            pl.BlockSpec((1, DH), lambda i: (0, 0)),
        ],
        out_specs=[
            pl.BlockSpec((NB, DH), lambda i: (i, 0)),
            pl.BlockSpec((NB, DH), lambda i: (i, 0)),
        ],
        out_shape=[
            jax.ShapeDtypeStruct((N, DH), jnp.float32),
            jax.ShapeDtypeStruct((N, DH), jnp.float32),
        ],
    )(h, wn, ws, bn)


def _sc_agg(hw, gate, src, dst):
    mesh = plsc.VectorSubcoreMesh(core_axis_name="c", subcore_axis_name="s")

    @functools.partial(
        pl.kernel,
        out_type=jax.ShapeDtypeStruct((NC, N, DH), jnp.float32),
        mesh=mesh,
        scratch_types=[
            pltpu.VMEM((NBUF, C), jnp.int32),
            pltpu.VMEM((NBUF, C), jnp.int32),
            pltpu.VMEM((NBUF, C, DH), jnp.float32),
            pltpu.VMEM((NBUF, C, DH), jnp.float32),
            pltpu.VMEM((ZROWS, DH), jnp.float32),
            pltpu.VMEM_SHARED((N, DH), jnp.float32),
            pltpu.SemaphoreType.DMA,
            pltpu.SemaphoreType.DMA,
        ],
    )
    def k(hw_hbm, gate_hbm, src_hbm, dst_hbm, out_hbm,
          sidx, didx, rows, grows, zbuf, aggsh, gsem, ssem):
        c = lax.axis_index("c")
        s = lax.axis_index("s")
        wid = c * NS + s
        zero = jnp.zeros((LANES,), jnp.float32)

        def zrow(i, carry):
            for jj in range(DH // LANES):
                zbuf[i, pl.ds(jj * LANES, LANES)] = zero
            return carry

        lax.fori_loop(0, ZROWS, zrow, 0)
        start = s * TROWS
        for kk in range(TROWS // ZROWS):
            pltpu.sync_copy(zbuf, aggsh.at[pl.ds(start + kk * ZROWS, ZROWS)])

        @pl.when(s == NS - 1)
        def _():
            pltpu.sync_copy(zbuf.at[pl.ds(0, 16)], aggsh.at[pl.ds(N - 16, 16)])

        plsc.subcore_barrier()

        def prep_chunk(g, b):
            base = wid * EPT + g * C
            sb = sidx.at[b]
            db = didx.at[b]
            rb = rows.at[b]
            gb = grows.at[b]
            pltpu.sync_copy(src_hbm.at[pl.ds(base, C)], sb)
            pltpu.sync_copy(dst_hbm.at[pl.ds(base, C)], db)
            pltpu.async_copy(hw_hbm.at[sb], rb, gsem).wait()
            pltpu.sync_copy(gate_hbm.at[pl.ds(base, C)], gb)

            def mul(i, cc):
                for jj in range(DH // LANES):
                    sl = pl.ds(jj * LANES, LANES)
                    rb[i, sl] = rb[i, sl] * gb[i, sl]
                return cc

            lax.fori_loop(0, C, mul, 0)

        def super_chunk(gg, carry):
            for b in range(NBUF):
                prep_chunk(gg * NBUF + b, b)
            for t in range(NS):
                @pl.when(s == t)
                def _():
                    for b in range(NBUF):
                        pltpu.async_copy(rows.at[b], aggsh.at[didx.at[b]], ssem,
                                         add=True).wait()
                plsc.subcore_barrier()
            return carry

        lax.fori_loop(0, NSUPER, super_chunk, 0)
        prep_chunk(NCHUNK - 1, 0)
        for t in range(NS):
            @pl.when(s == t)
            def _():
                pltpu.async_copy(rows.at[0], aggsh.at[didx.at[0]], ssem,
                                 add=True).wait()
            plsc.subcore_barrier()
        plsc.subcore_barrier()
        for kk in range(TROWS // ZROWS):
            pltpu.sync_copy(aggsh.at[pl.ds(start + kk * ZROWS, ZROWS)],
                            out_hbm.at[c, pl.ds(start + kk * ZROWS, ZROWS)])

        @pl.when(s == NS - 1)
        def _():
            pltpu.sync_copy(aggsh.at[pl.ds(N - 16, 16)],
                            out_hbm.at[c, pl.ds(N - 16, 16)])

    return k(hw, gate, src, dst)


def _combine_body(agg_ref, hs_ref, b_ref, h_ref, s_ref, c_ref):
    i = pl.program_id(0)
    h = jnp.maximum(agg_ref[0] + agg_ref[1] + hs_ref[...], 0.0)
    h_ref[...] = h
    bvec = b_ref[0, 0, :]
    iota = lax.broadcasted_iota(jnp.int32, (NB, B), 1)
    onehot = (bvec[:, None] == iota).astype(jnp.float32)
    ps = lax.dot_general(onehot, h, (((0,), (0,)), ((), ())),
                         preferred_element_type=jnp.float32,
                         precision=lax.Precision.HIGHEST)
    pc = lax.dot_general(onehot, jnp.ones_like(h), (((0,), (0,)), ((), ())),
                         preferred_element_type=jnp.float32,
                         precision=lax.Precision.HIGHEST)

    @pl.when(i == 0)
    def _():
        s_ref[...] = ps
        c_ref[...] = pc

    @pl.when(i > 0)
    def _():
        s_ref[...] = s_ref[...] + ps
        c_ref[...] = c_ref[...] + pc


def _combine(agg, hs, batch3):
    return pl.pallas_call(
        _combine_body,
        grid=(NGRID,),
        in_specs=[
            pl.BlockSpec((NC, NB, DH), lambda i: (0, i, 0)),
            pl.BlockSpec((NB, DH), lambda i: (i, 0)),
            pl.BlockSpec((1, 1, NB), lambda i: (i, 0, 0)),
        ],
        out_specs=[
            pl.BlockSpec((NB, DH), lambda i: (i, 0)),
            pl.BlockSpec((B, DH), lambda i: (0, 0)),
            pl.BlockSpec((B, DH), lambda i: (0, 0)),
        ],
        out_shape=[
            jax.ShapeDtypeStruct((N, DH), jnp.float32),
            jax.ShapeDtypeStruct((B, DH), jnp.float32),
            jax.ShapeDtypeStruct((B, DH), jnp.float32),
        ],
    )(agg, hs, batch3)


def _tail_body(xs_ref, cnt_ref, wx_ref, wh_ref, b_ref, att_ref,
               w1_ref, b1_ref, w2_ref, b2_ref, o_ref):
    denom = jnp.maximum(cnt_ref[...], 1.0)
    xs = [xs_ref[l] / denom for l in range(LAYERS)]
    wx = wx_ref[...]
    wh = wh_ref[...]
    bb = b_ref[...]
    hp = jnp.zeros((B, DH), jnp.float32)
    cp = jnp.zeros((B, DH), jnp.float32)
    hs = []
    for l in range(LAYERS):
        z = (jnp.dot(xs[l], wx, preferred_element_type=jnp.float32)
             + jnp.dot(hp, wh, preferred_element_type=jnp.float32) + bb)
        zi = z[:, 0:DH]
        zf = z[:, DH:2 * DH]
        zg = z[:, 2 * DH:3 * DH]
        zo = z[:, 3 * DH:4 * DH]
        cp = jax.nn.sigmoid(zf) * cp + jax.nn.sigmoid(zi) * jnp.tanh(zg)
        hp = jax.nn.sigmoid(zo) * jnp.tanh(cp)
        hs.append(hp)
    att = att_ref[...]
    scores = [jnp.sum(h * att, axis=1, keepdims=True) for h in hs]
    m = scores[0]
    for sc in scores[1:]:
        m = jnp.maximum(m, sc)
    exps = [jnp.exp(sc - m) for sc in scores]
    tot = exps[0]
    for ex in exps[1:]:
        tot = tot + ex
    out = jnp.zeros((B, DH), jnp.float32)
    for l in range(LAYERS):
        out = out + (exps[l] / tot) * xs[l]
    out = jnp.maximum(
        jnp.dot(out, w1_ref[...], preferred_element_type=jnp.float32) + b1_ref[...],
        0.0)
    out = jnp.dot(out, w2_ref[...], preferred_element_type=jnp.float32) + b2_ref[...]
    mx = jnp.max(out, axis=1, keepdims=True)
    lse = jnp.log(jnp.sum(jnp.exp(out - mx), axis=1, keepdims=True)) + mx
    o_ref[...] = out - lse


def _tail(xs, cnt, lstm, fc1_w, fc1_b, fc2_w, fc2_b):
    ins = [xs, cnt, lstm["Wx"], lstm["Wh"], lstm["b"].reshape(1, -1),
           lstm["att"].reshape(1, -1), fc1_w, fc1_b.reshape(1, -1),
           fc2_w, fc2_b.reshape(1, -1)]
    return pl.pallas_call(
        _tail_body,
        out_shape=jax.ShapeDtypeStruct((B, DH), jnp.float32),
    )(*ins)


def kernel(x, edge_index, edge_inform, batch, params):
    src = edge_index[0]
    dst = edge_index[1]
    blocks = params["blocks"]
    gates = _compute_gates(edge_inform, blocks)
    batch3 = batch.reshape(NGRID, 1, NB).astype(jnp.int32)
    h = x
    sums = []
    cnt = None
    for l in range(LAYERS):
        blk = blocks[l]
        hw, hs = _node_mm(h, blk["Wn"], blk["Wself"], blk["bn"].reshape(1, DH))
        agg = _sc_agg(hw, gates[l], src, dst)
        h, ssum, cl = _combine(agg, hs, batch3)
        sums.append(ssum)
        if cnt is None:
            cnt = cl
    xs = jnp.stack(sums, axis=0)
    return _tail(xs, cnt, params["lstm"], params["fc1_W"], params["fc1_b"],
                 params["fc2_W"], params["fc2_b"])


# speedup vs baseline: 2.8882x; 2.5900x over previous
import functools

import jax
import jax.numpy as jnp
from jax import lax
from jax.experimental import pallas as pl
from jax.experimental.pallas import tpu as pltpu
from jax.experimental.pallas import tpu_sc as plsc

N = 10000
E = 320000
B = 64
DH = 128
LAYERS = 6

NC = 2
NS = 16
LANES = 16

EB = 2560
C = 80
NBUF = 2
EPT = E // (NC * NS)
NCHUNK = EPT // C
NSUPER = NCHUNK // NBUF
TROWS = 624
ZROWS = 8
NB = 2000
NGRID = N // NB


def _gates_body(e_ref, *refs):
    wrefs = refs[:3 * LAYERS]
    orefs = refs[3 * LAYERS:]
    eb = e_ref[...]
    for l in range(LAYERS):
        we, be, wg = wrefs[3 * l], wrefs[3 * l + 1], wrefs[3 * l + 2]
        eh = jnp.maximum(
            jnp.dot(eb, we[...], preferred_element_type=jnp.float32) + be[...], 0.0)
        orefs[l][...] = jax.nn.sigmoid(
            jnp.dot(eh, wg[...], preferred_element_type=jnp.float32))
        eb = eh


def _compute_gates(e, blocks):
    ins = [e]
    in_specs = [pl.BlockSpec((EB, e.shape[1]), lambda i: (i, 0))]
    for blk in blocks:
        for w in (blk["We"], blk["be"].reshape(1, -1), blk["Wg"]):
            ins.append(w)
            in_specs.append(pl.BlockSpec(w.shape, lambda i: (0, 0)))
    return pl.pallas_call(
        _gates_body,
        grid=(E // EB,),
        in_specs=in_specs,
        out_specs=[pl.BlockSpec((EB, DH), lambda i: (i, 0))] * LAYERS,
        out_shape=[jax.ShapeDtypeStruct((E, DH), jnp.float32)] * LAYERS,
    )(*ins)


def _node_mm_body(h_ref, wn_ref, ws_ref, bn_ref, hw_ref, hs_ref):
    h = h_ref[...]
    hw_ref[...] = jnp.dot(h, wn_ref[...], preferred_element_type=jnp.float32)
    hs_ref[...] = (
        jnp.dot(h, ws_ref[...], preferred_element_type=jnp.float32) + bn_ref[...])


def _node_mm(h, wn, ws, bn):
    d_in = h.shape[1]
    return pl.pallas_call(
        _node_mm_body,
        grid=(NGRID,),
        in_specs=[
            pl.BlockSpec((NB, d_in), lambda i: (i, 0)),
            pl.BlockSpec((d_in, DH), lambda i: (0, 0)),
            pl.BlockSpec((d_in, DH), lambda i: (0, 0)),
            pl.BlockSpec((1, DH), lambda i: (0, 0)),
        ],
        out_specs=[
            pl.BlockSpec((NB, DH), lambda i: (i, 0)),
            pl.BlockSpec((NB, DH), lambda i: (i, 0)),
        ],
        out_shape=[
            jax.ShapeDtypeStruct((N, DH), jnp.float32),
            jax.ShapeDtypeStruct((N, DH), jnp.float32),
        ],
    )(h, wn, ws, bn)


def _sc_agg(hw, gate, src, dst):
    mesh = plsc.VectorSubcoreMesh(core_axis_name="c", subcore_axis_name="s")

    @functools.partial(
        pl.kernel,
        out_type=jax.ShapeDtypeStruct((NC, N, DH), jnp.float32),
        mesh=mesh,
        scratch_types=[
            pltpu.VMEM((NBUF, C), jnp.int32),
            pltpu.VMEM((NBUF, C), jnp.int32),
            pltpu.VMEM((NBUF, C, DH), jnp.float32),
            pltpu.VMEM((NBUF, C, DH), jnp.float32),
            pltpu.VMEM((ZROWS, DH), jnp.float32),
            pltpu.VMEM_SHARED((N, DH), jnp.float32),
            pltpu.SemaphoreType.DMA,
            pltpu.SemaphoreType.DMA,
        ],
    )
    def k(hw_hbm, gate_hbm, src_hbm, dst_hbm, out_hbm,
          sidx, didx, rows, grows, zbuf, aggsh, gsem, ssem):
        c = lax.axis_index("c")
        s = lax.axis_index("s")
        wid = c * NS + s
        zero = jnp.zeros((LANES,), jnp.float32)

        def zrow(i, carry):
            for jj in range(DH // LANES):
                zbuf[i, pl.ds(jj * LANES, LANES)] = zero
            return carry

        lax.fori_loop(0, ZROWS, zrow, 0)
        start = s * TROWS
        for kk in range(TROWS // ZROWS):
            pltpu.sync_copy(zbuf, aggsh.at[pl.ds(start + kk * ZROWS, ZROWS)])

        @pl.when(s == NS - 1)
        def _():
            pltpu.sync_copy(zbuf.at[pl.ds(0, 16)], aggsh.at[pl.ds(N - 16, 16)])

        plsc.subcore_barrier()

        def prep_chunk(g, b):
            base = wid * EPT + g * C
            sb = sidx.at[b]
            db = didx.at[b]
            rb = rows.at[b]
            gb = grows.at[b]
            pltpu.sync_copy(src_hbm.at[pl.ds(base, C)], sb)
            pltpu.sync_copy(dst_hbm.at[pl.ds(base, C)], db)
            g = pltpu.async_copy(hw_hbm.at[sb], rb, gsem)
            pltpu.sync_copy(gate_hbm.at[pl.ds(base, C)], gb)
            g.wait()

            def mul(i, cc):
                for jj in range(DH // LANES):
                    sl = pl.ds(jj * LANES, LANES)
                    rb[i, sl] = rb[i, sl] * gb[i, sl]
                return cc

            lax.fori_loop(0, C, mul, 0)

        def super_chunk(gg, carry):
            for b in range(NBUF):
                prep_chunk(gg * NBUF + b, b)
            for b in range(NBUF):
                pltpu.async_copy(rows.at[b], aggsh.at[didx.at[b]], ssem,
                                 add=True).wait()
            return carry

        lax.fori_loop(0, NSUPER, super_chunk, 0)
        prep_chunk(NCHUNK - 1, 0)
        pltpu.async_copy(rows.at[0], aggsh.at[didx.at[0]], ssem,
                         add=True).wait()
        plsc.subcore_barrier()
        for kk in range(TROWS // ZROWS):
            pltpu.sync_copy(aggsh.at[pl.ds(start + kk * ZROWS, ZROWS)],
                            out_hbm.at[c, pl.ds(start + kk * ZROWS, ZROWS)])

        @pl.when(s == NS - 1)
        def _():
            pltpu.sync_copy(aggsh.at[pl.ds(N - 16, 16)],
                            out_hbm.at[c, pl.ds(N - 16, 16)])

    return k(hw, gate, src, dst)


def _combine_body(agg_ref, hs_ref, b_ref, h_ref, s_ref, c_ref):
    i = pl.program_id(0)
    h = jnp.maximum(agg_ref[0] + agg_ref[1] + hs_ref[...], 0.0)
    h_ref[...] = h
    bvec = b_ref[0, 0, :]
    iota = lax.broadcasted_iota(jnp.int32, (NB, B), 1)
    onehot = (bvec[:, None] == iota).astype(jnp.float32)
    ps = lax.dot_general(onehot, h, (((0,), (0,)), ((), ())),
                         preferred_element_type=jnp.float32,
                         precision=lax.Precision.HIGHEST)
    pc = lax.dot_general(onehot, jnp.ones_like(h), (((0,), (0,)), ((), ())),
                         preferred_element_type=jnp.float32,
                         precision=lax.Precision.HIGHEST)

    @pl.when(i == 0)
    def _():
        s_ref[...] = ps
        c_ref[...] = pc

    @pl.when(i > 0)
    def _():
        s_ref[...] = s_ref[...] + ps
        c_ref[...] = c_ref[...] + pc


def _combine(agg, hs, batch3):
    return pl.pallas_call(
        _combine_body,
        grid=(NGRID,),
        in_specs=[
            pl.BlockSpec((NC, NB, DH), lambda i: (0, i, 0)),
            pl.BlockSpec((NB, DH), lambda i: (i, 0)),
            pl.BlockSpec((1, 1, NB), lambda i: (i, 0, 0)),
        ],
        out_specs=[
            pl.BlockSpec((NB, DH), lambda i: (i, 0)),
            pl.BlockSpec((B, DH), lambda i: (0, 0)),
            pl.BlockSpec((B, DH), lambda i: (0, 0)),
        ],
        out_shape=[
            jax.ShapeDtypeStruct((N, DH), jnp.float32),
            jax.ShapeDtypeStruct((B, DH), jnp.float32),
            jax.ShapeDtypeStruct((B, DH), jnp.float32),
        ],
    )(agg, hs, batch3)


def _tail_body(xs_ref, cnt_ref, wx_ref, wh_ref, b_ref, att_ref,
               w1_ref, b1_ref, w2_ref, b2_ref, o_ref):
    denom = jnp.maximum(cnt_ref[...], 1.0)
    xs = [xs_ref[l] / denom for l in range(LAYERS)]
    wx = wx_ref[...]
    wh = wh_ref[...]
    bb = b_ref[...]
    hp = jnp.zeros((B, DH), jnp.float32)
    cp = jnp.zeros((B, DH), jnp.float32)
    hs = []
    for l in range(LAYERS):
        z = (jnp.dot(xs[l], wx, preferred_element_type=jnp.float32)
             + jnp.dot(hp, wh, preferred_element_type=jnp.float32) + bb)
        zi = z[:, 0:DH]
        zf = z[:, DH:2 * DH]
        zg = z[:, 2 * DH:3 * DH]
        zo = z[:, 3 * DH:4 * DH]
        cp = jax.nn.sigmoid(zf) * cp + jax.nn.sigmoid(zi) * jnp.tanh(zg)
        hp = jax.nn.sigmoid(zo) * jnp.tanh(cp)
        hs.append(hp)
    att = att_ref[...]
    scores = [jnp.sum(h * att, axis=1, keepdims=True) for h in hs]
    m = scores[0]
    for sc in scores[1:]:
        m = jnp.maximum(m, sc)
    exps = [jnp.exp(sc - m) for sc in scores]
    tot = exps[0]
    for ex in exps[1:]:
        tot = tot + ex
    out = jnp.zeros((B, DH), jnp.float32)
    for l in range(LAYERS):
        out = out + (exps[l] / tot) * xs[l]
    out = jnp.maximum(
        jnp.dot(out, w1_ref[...], preferred_element_type=jnp.float32) + b1_ref[...],
        0.0)
    out = jnp.dot(out, w2_ref[...], preferred_element_type=jnp.float32) + b2_ref[...]
    mx = jnp.max(out, axis=1, keepdims=True)
    lse = jnp.log(jnp.sum(jnp.exp(out - mx), axis=1, keepdims=True)) + mx
    o_ref[...] = out - lse


def _tail(xs, cnt, lstm, fc1_w, fc1_b, fc2_w, fc2_b):
    ins = [xs, cnt, lstm["Wx"], lstm["Wh"], lstm["b"].reshape(1, -1),
           lstm["att"].reshape(1, -1), fc1_w, fc1_b.reshape(1, -1),
           fc2_w, fc2_b.reshape(1, -1)]
    return pl.pallas_call(
        _tail_body,
        out_shape=jax.ShapeDtypeStruct((B, DH), jnp.float32),
    )(*ins)


def kernel(x, edge_index, edge_inform, batch, params):
    src = edge_index[0]
    dst = edge_index[1]
    blocks = params["blocks"]
    gates = _compute_gates(edge_inform, blocks)
    batch3 = batch.reshape(NGRID, 1, NB).astype(jnp.int32)
    h = x
    sums = []
    cnt = None
    for l in range(LAYERS):
        blk = blocks[l]
        hw, hs = _node_mm(h, blk["Wn"], blk["Wself"], blk["bn"].reshape(1, DH))
        agg = _sc_agg(hw, gates[l], src, dst)
        h, ssum, cl = _combine(agg, hs, batch3)
        sums.append(ssum)
        if cnt is None:
            cnt = cl
    xs = jnp.stack(sums, axis=0)
    return _tail(xs, cnt, params["lstm"], params["fc1_W"], params["fc1_b"],
                 params["fc2_W"], params["fc2_b"])


# speedup vs baseline: 3.7629x; 1.3029x over previous
import functools

import jax
import jax.numpy as jnp
from jax import lax
from jax.experimental import pallas as pl
from jax.experimental.pallas import tpu as pltpu
from jax.experimental.pallas import tpu_sc as plsc

N = 10000
E = 320000
B = 64
DH = 128
LAYERS = 6

NC = 2
NS = 16
LANES = 16

EB = 2560
C = 80
NBUF = 2
EPT = E // (NC * NS)
NCHUNK = EPT // C
NSUPER = NCHUNK // NBUF
TROWS = 624
ZROWS = 8
NB = 2000
NGRID = N // NB


def _gates_body(e_ref, *refs):
    wrefs = refs[:3 * LAYERS]
    orefs = refs[3 * LAYERS:]
    eb = e_ref[...]
    for l in range(LAYERS):
        we, be, wg = wrefs[3 * l], wrefs[3 * l + 1], wrefs[3 * l + 2]
        eh = jnp.maximum(
            jnp.dot(eb, we[...], preferred_element_type=jnp.float32) + be[...], 0.0)
        orefs[l][...] = jax.nn.sigmoid(
            jnp.dot(eh, wg[...], preferred_element_type=jnp.float32))
        eb = eh


def _compute_gates(e, blocks):
    ins = [e]
    in_specs = [pl.BlockSpec((EB, e.shape[1]), lambda i: (i, 0))]
    for blk in blocks:
        for w in (blk["We"], blk["be"].reshape(1, -1), blk["Wg"]):
            ins.append(w)
            in_specs.append(pl.BlockSpec(w.shape, lambda i: (0, 0)))
    return pl.pallas_call(
        _gates_body,
        grid=(E // EB,),
        in_specs=in_specs,
        out_specs=[pl.BlockSpec((EB, DH), lambda i: (i, 0))] * LAYERS,
        out_shape=[jax.ShapeDtypeStruct((E, DH), jnp.float32)] * LAYERS,
    )(*ins)


def _node_mm_body(h_ref, wn_ref, ws_ref, bn_ref, hw_ref, hs_ref):
    h = h_ref[...]
    hw_ref[...] = jnp.dot(h, wn_ref[...], preferred_element_type=jnp.float32)
    hs_ref[...] = (
        jnp.dot(h, ws_ref[...], preferred_element_type=jnp.float32) + bn_ref[...])


def _node_mm(h, wn, ws, bn):
    d_in = h.shape[1]
    return pl.pallas_call(
        _node_mm_body,
        grid=(NGRID,),
        in_specs=[
            pl.BlockSpec((NB, d_in), lambda i: (i, 0)),
            pl.BlockSpec((d_in, DH), lambda i: (0, 0)),
            pl.BlockSpec((d_in, DH), lambda i: (0, 0)),
            pl.BlockSpec((1, DH), lambda i: (0, 0)),
        ],
        out_specs=[
            pl.BlockSpec((NB, DH), lambda i: (i, 0)),
            pl.BlockSpec((NB, DH), lambda i: (i, 0)),
        ],
        out_shape=[
            jax.ShapeDtypeStruct((N, DH), jnp.float32),
            jax.ShapeDtypeStruct((N, DH), jnp.float32),
        ],
    )(h, wn, ws, bn)


def _sc_agg(hw, gate, src, dst):
    mesh = plsc.VectorSubcoreMesh(core_axis_name="c", subcore_axis_name="s")

    @functools.partial(
        pl.kernel,
        out_type=jax.ShapeDtypeStruct((NC, N, DH), jnp.float32),
        mesh=mesh,
        scratch_types=[
            pltpu.VMEM((NBUF, C), jnp.int32),
            pltpu.VMEM((NBUF, C), jnp.int32),
            pltpu.VMEM((NBUF, C, DH), jnp.float32),
            pltpu.VMEM((NBUF, C, DH), jnp.float32),
            pltpu.VMEM((ZROWS, DH), jnp.float32),
            pltpu.VMEM_SHARED((N, DH), jnp.float32),
            pltpu.SemaphoreType.DMA,
            pltpu.SemaphoreType.DMA,
            pltpu.SemaphoreType.DMA,
            pltpu.SemaphoreType.DMA,
            pltpu.SemaphoreType.DMA,
            pltpu.SemaphoreType.DMA,
            pltpu.SemaphoreType.DMA,
            pltpu.SemaphoreType.DMA,
            pltpu.SemaphoreType.DMA,
        ],
    )
    def k(hw_hbm, gate_hbm, src_hbm, dst_hbm, out_hbm,
          sidx, didx, rows, grows, zbuf, aggsh,
          ss0, ss1, sd0, sd1, sg0, sg1, st0, st1, ssem):
        c = lax.axis_index("c")
        s = lax.axis_index("s")
        wid = c * NS + s
        zero = jnp.zeros((LANES,), jnp.float32)

        def zrow(i, carry):
            for jj in range(DH // LANES):
                zbuf[i, pl.ds(jj * LANES, LANES)] = zero
            return carry

        lax.fori_loop(0, ZROWS, zrow, 0)
        start = s * TROWS
        for kk in range(TROWS // ZROWS):
            pltpu.sync_copy(zbuf, aggsh.at[pl.ds(start + kk * ZROWS, ZROWS)])

        @pl.when(s == NS - 1)
        def _():
            pltpu.sync_copy(zbuf.at[pl.ds(0, 16)], aggsh.at[pl.ds(N - 16, 16)])

        plsc.subcore_barrier()

        def mul_rows(b):
            rb = rows.at[b]
            gb = grows.at[b]

            def mul(i, cc):
                for r in range(4):
                    for jj in range(DH // LANES):
                        sl = pl.ds(jj * LANES, LANES)
                        rb[i * 4 + r, sl] = rb[i * 4 + r, sl] * gb[i * 4 + r, sl]
                return cc

            lax.fori_loop(0, C // 4, mul, 0)

        def super_chunk(gg, carry):
            base0 = wid * EPT + (gg * NBUF) * C
            base1 = base0 + C
            hs0 = pltpu.async_copy(src_hbm.at[pl.ds(base0, C)], sidx.at[0], ss0)
            hd0 = pltpu.async_copy(dst_hbm.at[pl.ds(base0, C)], didx.at[0], sd0)
            hs1 = pltpu.async_copy(src_hbm.at[pl.ds(base1, C)], sidx.at[1], ss1)
            hd1 = pltpu.async_copy(dst_hbm.at[pl.ds(base1, C)], didx.at[1], sd1)
            hs0.wait()
            hg0 = pltpu.async_copy(hw_hbm.at[sidx.at[0]], rows.at[0], sg0)
            ht0 = pltpu.async_copy(gate_hbm.at[pl.ds(base0, C)], grows.at[0], st0)
            hs1.wait()
            hg1 = pltpu.async_copy(hw_hbm.at[sidx.at[1]], rows.at[1], sg1)
            ht1 = pltpu.async_copy(gate_hbm.at[pl.ds(base1, C)], grows.at[1], st1)
            hg0.wait()
            ht0.wait()
            mul_rows(0)
            hd0.wait()
            sc0 = pltpu.async_copy(rows.at[0], aggsh.at[didx.at[0]], ssem,
                                   add=True)
            hg1.wait()
            ht1.wait()
            mul_rows(1)
            sc0.wait()
            hd1.wait()
            pltpu.async_copy(rows.at[1], aggsh.at[didx.at[1]], ssem,
                             add=True).wait()
            return carry

        lax.fori_loop(0, NSUPER, super_chunk, 0)
        tbase = wid * EPT + (NCHUNK - 1) * C
        pltpu.sync_copy(src_hbm.at[pl.ds(tbase, C)], sidx.at[0])
        pltpu.sync_copy(dst_hbm.at[pl.ds(tbase, C)], didx.at[0])
        hgt = pltpu.async_copy(hw_hbm.at[sidx.at[0]], rows.at[0], sg0)
        pltpu.sync_copy(gate_hbm.at[pl.ds(tbase, C)], grows.at[0])
        hgt.wait()
        mul_rows(0)
        pltpu.async_copy(rows.at[0], aggsh.at[didx.at[0]], ssem,
                         add=True).wait()
        plsc.subcore_barrier()
        for kk in range(TROWS // ZROWS):
            pltpu.sync_copy(aggsh.at[pl.ds(start + kk * ZROWS, ZROWS)],
                            out_hbm.at[c, pl.ds(start + kk * ZROWS, ZROWS)])

        @pl.when(s == NS - 1)
        def _():
            pltpu.sync_copy(aggsh.at[pl.ds(N - 16, 16)],
                            out_hbm.at[c, pl.ds(N - 16, 16)])

    return k(hw, gate, src, dst)


def _combine_body(agg_ref, hs_ref, b_ref, h_ref, s_ref, c_ref):
    i = pl.program_id(0)
    h = jnp.maximum(agg_ref[0] + agg_ref[1] + hs_ref[...], 0.0)
    h_ref[...] = h
    bvec = b_ref[0, 0, :]
    iota = lax.broadcasted_iota(jnp.int32, (NB, B), 1)
    onehot = (bvec[:, None] == iota).astype(jnp.float32)
    ps = lax.dot_general(onehot, h, (((0,), (0,)), ((), ())),
                         preferred_element_type=jnp.float32,
                         precision=lax.Precision.HIGHEST)
    pc = lax.dot_general(onehot, jnp.ones_like(h), (((0,), (0,)), ((), ())),
                         preferred_element_type=jnp.float32,
                         precision=lax.Precision.HIGHEST)

    @pl.when(i == 0)
    def _():
        s_ref[...] = ps
        c_ref[...] = pc

    @pl.when(i > 0)
    def _():
        s_ref[...] = s_ref[...] + ps
        c_ref[...] = c_ref[...] + pc


def _combine(agg, hs, batch3):
    return pl.pallas_call(
        _combine_body,
        grid=(NGRID,),
        in_specs=[
            pl.BlockSpec((NC, NB, DH), lambda i: (0, i, 0)),
            pl.BlockSpec((NB, DH), lambda i: (i, 0)),
            pl.BlockSpec((1, 1, NB), lambda i: (i, 0, 0)),
        ],
        out_specs=[
            pl.BlockSpec((NB, DH), lambda i: (i, 0)),
            pl.BlockSpec((B, DH), lambda i: (0, 0)),
            pl.BlockSpec((B, DH), lambda i: (0, 0)),
        ],
        out_shape=[
            jax.ShapeDtypeStruct((N, DH), jnp.float32),
            jax.ShapeDtypeStruct((B, DH), jnp.float32),
            jax.ShapeDtypeStruct((B, DH), jnp.float32),
        ],
    )(agg, hs, batch3)


def _tail_body(xs_ref, cnt_ref, wx_ref, wh_ref, b_ref, att_ref,
               w1_ref, b1_ref, w2_ref, b2_ref, o_ref):
    denom = jnp.maximum(cnt_ref[...], 1.0)
    xs = [xs_ref[l] / denom for l in range(LAYERS)]
    wx = wx_ref[...]
    wh = wh_ref[...]
    bb = b_ref[...]
    hp = jnp.zeros((B, DH), jnp.float32)
    cp = jnp.zeros((B, DH), jnp.float32)
    hs = []
    for l in range(LAYERS):
        z = (jnp.dot(xs[l], wx, preferred_element_type=jnp.float32)
             + jnp.dot(hp, wh, preferred_element_type=jnp.float32) + bb)
        zi = z[:, 0:DH]
        zf = z[:, DH:2 * DH]
        zg = z[:, 2 * DH:3 * DH]
        zo = z[:, 3 * DH:4 * DH]
        cp = jax.nn.sigmoid(zf) * cp + jax.nn.sigmoid(zi) * jnp.tanh(zg)
        hp = jax.nn.sigmoid(zo) * jnp.tanh(cp)
        hs.append(hp)
    att = att_ref[...]
    scores = [jnp.sum(h * att, axis=1, keepdims=True) for h in hs]
    m = scores[0]
    for sc in scores[1:]:
        m = jnp.maximum(m, sc)
    exps = [jnp.exp(sc - m) for sc in scores]
    tot = exps[0]
    for ex in exps[1:]:
        tot = tot + ex
    out = jnp.zeros((B, DH), jnp.float32)
    for l in range(LAYERS):
        out = out + (exps[l] / tot) * xs[l]
    out = jnp.maximum(
        jnp.dot(out, w1_ref[...], preferred_element_type=jnp.float32) + b1_ref[...],
        0.0)
    out = jnp.dot(out, w2_ref[...], preferred_element_type=jnp.float32) + b2_ref[...]
    mx = jnp.max(out, axis=1, keepdims=True)
    lse = jnp.log(jnp.sum(jnp.exp(out - mx), axis=1, keepdims=True)) + mx
    o_ref[...] = out - lse


def _tail(xs, cnt, lstm, fc1_w, fc1_b, fc2_w, fc2_b):
    ins = [xs, cnt, lstm["Wx"], lstm["Wh"], lstm["b"].reshape(1, -1),
           lstm["att"].reshape(1, -1), fc1_w, fc1_b.reshape(1, -1),
           fc2_w, fc2_b.reshape(1, -1)]
    return pl.pallas_call(
        _tail_body,
        out_shape=jax.ShapeDtypeStruct((B, DH), jnp.float32),
    )(*ins)


def kernel(x, edge_index, edge_inform, batch, params):
    src = edge_index[0]
    dst = edge_index[1]
    blocks = params["blocks"]
    gates = _compute_gates(edge_inform, blocks)
    batch3 = batch.reshape(NGRID, 1, NB).astype(jnp.int32)
    h = x
    sums = []
    cnt = None
    for l in range(LAYERS):
        blk = blocks[l]
        hw, hs = _node_mm(h, blk["Wn"], blk["Wself"], blk["bn"].reshape(1, DH))
        agg = _sc_agg(hw, gates[l], src, dst)
        h, ssum, cl = _combine(agg, hs, batch3)
        sums.append(ssum)
        if cnt is None:
            cnt = cl
    xs = jnp.stack(sums, axis=0)
    return _tail(xs, cnt, params["lstm"], params["fc1_W"], params["fc1_b"],
                 params["fc2_W"], params["fc2_b"])
